# Initial kernel scaffold; baseline (speedup 1.0000x reference)
import jax
import jax.numpy as jnp
from jax.experimental import pallas as pl


def kernel(X, params):
    raise NotImplementedError("write your pallas kernel here")


# speedup vs baseline: 572.0697x; 572.0697x over previous
import functools

import jax
import jax.numpy as jnp
from jax import lax
from jax.experimental import pallas as pl
from jax.experimental.pallas import tpu as pltpu
from jax.experimental.pallas import tpu_sc as plsc

_ARITY = 4
_GROUPS = 256
_OUT_DIM = 4
_BATCH = 16384
_ROW = _GROUPS * _ARITY
_N = _BATCH * _ROW
_NC = 2
_NS = 16
_NW = _NC * _NS
_SPAN = _N // _NW
_CHUNK = 16 * _ROW
_NCHUNK = _SPAN // _CHUNK
_PTWORDS = _OUT_DIM * _GROUPS * 16
_P15WORDS = _OUT_DIM * _GROUPS


def _sc_body(x_hbm, pt_hbm, p15_hbm, o_hbm, pt, p15, xv, ov):
    wid = lax.axis_index("s") * _NC + lax.axis_index("c")
    pltpu.sync_copy(pt_hbm, pt)
    pltpu.sync_copy(p15_hbm, p15)

    iota = jnp.arange(16, dtype=jnp.int32)
    lane4 = iota * 4
    lane16 = iota * 16
    hi_mask = jnp.int32(-16)

    base = wid * _SPAN

    def chunk_body(c, carry):
        cbase = base + c * _CHUNK
        pltpu.sync_copy(x_hbm.at[pl.ds(cbase, _CHUNK)], xv)

        def blk(u, carry2):
            for t_off in range(2):
                t = u * 2 + t_off
                i0 = t * 64 + lane4
                gb = t & 15
                gvec = gb * 16 + iota
                gv16 = gb * 256 + lane16
                r0 = plsc.load_gather(xv, [i0])
                r1 = plsc.load_gather(xv, [i0 + 1])
                r2 = plsc.load_gather(xv, [i0 + 2])
                r3 = plsc.load_gather(xv, [i0 + 3])
                v0 = lax.bitcast_convert_type(
                    (lax.bitcast_convert_type(r0, jnp.int32) & hi_mask) | 1,
                    jnp.float32)
                v1 = lax.bitcast_convert_type(
                    (lax.bitcast_convert_type(r1, jnp.int32) & hi_mask) | 2,
                    jnp.float32)
                v2 = lax.bitcast_convert_type(
                    (lax.bitcast_convert_type(r2, jnp.int32) & hi_mask) | 4,
                    jnp.float32)
                v3 = lax.bitcast_convert_type(
                    (lax.bitcast_convert_type(r3, jnp.int32) & hi_mask) | 8,
                    jnp.float32)
                a, b = jnp.minimum(v0, v1), jnp.maximum(v0, v1)
                c2_, d2 = jnp.minimum(v2, v3), jnp.maximum(v2, v3)
                f0, s2_ = jnp.minimum(a, c2_), jnp.maximum(a, c2_)
                s1_, f3 = jnp.minimum(b, d2), jnp.maximum(b, d2)
                f1 = jnp.minimum(s1_, s2_)
                f2 = jnp.maximum(s1_, s2_)
                s1 = lax.bitcast_convert_type(f1, jnp.int32)
                s2 = lax.bitcast_convert_type(f2, jnp.int32)
                s3 = lax.bitcast_convert_type(f3, jnp.int32)
                m3 = s3 & 15
                m2 = m3 + (s2 & 15)
                m1 = m2 + (s1 & 15)
                q1 = gv16 + ((m1 + gvec) & 15)
                q2 = gv16 + ((m2 + gvec) & 15)
                q3 = gv16 + ((m3 + gvec) & 15)
                c1 = f1 - f0
                c2c = f2 - f1
                c3 = f3 - f2
                for d in range(_OUT_DIM):
                    p15d = p15[pl.ds(d * _GROUPS + gb * 16, 16)]
                    acc0 = f0 * p15d + c1 * plsc.load_gather(pt, [q1 | (d * 4096)])
                    acc1 = c2c * plsc.load_gather(pt, [q2 | (d * 4096)]) + \
                        c3 * plsc.load_gather(pt, [q3 | (d * 4096)])
                    plsc.store_scatter(ov, [i0 + d], acc0 + acc1)
            return carry2

        lax.fori_loop(0, _CHUNK // 128, blk, 0)
        pltpu.sync_copy(ov, o_hbm.at[pl.ds(cbase, _CHUNK)])
        return carry

    lax.fori_loop(0, _NCHUNK, chunk_body, 0)


_sc_call = functools.partial(
    pl.kernel,
    mesh=plsc.VectorSubcoreMesh(core_axis_name="c", subcore_axis_name="s"),
    out_type=jax.ShapeDtypeStruct((_N,), jnp.float32),
    compiler_params=pltpu.CompilerParams(needs_layout_passes=False),
    scratch_types=[
        pltpu.VMEM((_PTWORDS,), jnp.float32),
        pltpu.VMEM((_P15WORDS,), jnp.float32),
        pltpu.VMEM((_CHUNK,), jnp.float32),
        pltpu.VMEM((_CHUNK,), jnp.float32),
    ],
)(_sc_body)


def kernel(X, params):
    g = jnp.arange(_GROUPS, dtype=jnp.int32)[:, None]
    s = jnp.arange(16, dtype=jnp.int32)[None, :]
    rolled = params[g, (s - g) % 16, :]
    pt = jnp.transpose(rolled, (2, 0, 1)).reshape(-1)
    p15 = params[:, 15, :].T.reshape(-1)
    out = _sc_call(X.reshape(-1), pt, p15)
    return out.reshape(X.shape[0], _ROW)


# speedup vs baseline: 776.1003x; 1.3567x over previous
import functools

import jax
import jax.numpy as jnp
from jax import lax
from jax.experimental import pallas as pl
from jax.experimental.pallas import tpu as pltpu
from jax.experimental.pallas import tpu_sc as plsc

_ARITY = 4
_GROUPS = 256
_OUT_DIM = 4
_BATCH = 16384
_ROW = _GROUPS * _ARITY
_N = _BATCH * _ROW
_NC = 2
_NS = 16
_NW = _NC * _NS
_SPAN = _N // _NW
_CHUNK = 16 * _ROW
_NCHUNK = _SPAN // _CHUNK
_PTWORDS = _OUT_DIM * _GROUPS * 16
_P15WORDS = _OUT_DIM * _GROUPS


def _sc_body(x_hbm, pt_hbm, p15_hbm, o_hbm, pt, p15, xv, ov):
    wid = lax.axis_index("s") * _NC + lax.axis_index("c")
    pltpu.sync_copy(pt_hbm, pt)
    pltpu.sync_copy(p15_hbm, p15)

    iota = jnp.arange(16, dtype=jnp.int32)
    lane4 = iota * 4
    lane16 = iota * 16
    hi_mask = jnp.int32(-16)

    base = wid * _SPAN

    def chunk_body(c, carry):
        cbase = base + c * _CHUNK
        pltpu.sync_copy(x_hbm.at[pl.ds(cbase, _CHUNK)], xv)

        @plsc.parallel_loop(0, _CHUNK // 64, unroll=4)
        def blk(t):
            if True:
                i0 = t * 64 + lane4
                gb = t & 15
                gvec = gb * 16 + iota
                gv16 = gb * 256 + lane16
                r0 = plsc.load_gather(xv, [i0])
                r1 = plsc.load_gather(xv, [i0 + 1])
                r2 = plsc.load_gather(xv, [i0 + 2])
                r3 = plsc.load_gather(xv, [i0 + 3])
                v0 = lax.bitcast_convert_type(
                    (lax.bitcast_convert_type(r0, jnp.int32) & hi_mask) | 1,
                    jnp.float32)
                v1 = lax.bitcast_convert_type(
                    (lax.bitcast_convert_type(r1, jnp.int32) & hi_mask) | 2,
                    jnp.float32)
                v2 = lax.bitcast_convert_type(
                    (lax.bitcast_convert_type(r2, jnp.int32) & hi_mask) | 4,
                    jnp.float32)
                v3 = lax.bitcast_convert_type(
                    (lax.bitcast_convert_type(r3, jnp.int32) & hi_mask) | 8,
                    jnp.float32)
                a, b = jnp.minimum(v0, v1), jnp.maximum(v0, v1)
                c2_, d2 = jnp.minimum(v2, v3), jnp.maximum(v2, v3)
                f0, s2_ = jnp.minimum(a, c2_), jnp.maximum(a, c2_)
                s1_, f3 = jnp.minimum(b, d2), jnp.maximum(b, d2)
                f1 = jnp.minimum(s1_, s2_)
                f2 = jnp.maximum(s1_, s2_)
                s1 = lax.bitcast_convert_type(f1, jnp.int32)
                s2 = lax.bitcast_convert_type(f2, jnp.int32)
                s3 = lax.bitcast_convert_type(f3, jnp.int32)
                m3 = s3 & 15
                m2 = m3 + (s2 & 15)
                m1 = m2 + (s1 & 15)
                q1 = gv16 + ((m1 + gvec) & 15)
                q2 = gv16 + ((m2 + gvec) & 15)
                q3 = gv16 + ((m3 + gvec) & 15)
                c1 = f1 - f0
                c2c = f2 - f1
                c3 = f3 - f2
                for d in range(_OUT_DIM):
                    p15d = p15[pl.ds(d * _GROUPS + gb * 16, 16)]
                    acc0 = f0 * p15d + c1 * plsc.load_gather(pt, [q1 | (d * 4096)])
                    acc1 = c2c * plsc.load_gather(pt, [q2 | (d * 4096)]) + \
                        c3 * plsc.load_gather(pt, [q3 | (d * 4096)])
                    plsc.store_scatter(ov, [i0 + d], acc0 + acc1)

        pltpu.sync_copy(ov, o_hbm.at[pl.ds(cbase, _CHUNK)])
        return carry

    lax.fori_loop(0, _NCHUNK, chunk_body, 0)


_sc_call = functools.partial(
    pl.kernel,
    mesh=plsc.VectorSubcoreMesh(core_axis_name="c", subcore_axis_name="s"),
    out_type=jax.ShapeDtypeStruct((_N,), jnp.float32),
    compiler_params=pltpu.CompilerParams(needs_layout_passes=False),
    scratch_types=[
        pltpu.VMEM((_PTWORDS,), jnp.float32),
        pltpu.VMEM((_P15WORDS,), jnp.float32),
        pltpu.VMEM((_CHUNK,), jnp.float32),
        pltpu.VMEM((_CHUNK,), jnp.float32),
    ],
)(_sc_body)


def kernel(X, params):
    g = jnp.arange(_GROUPS, dtype=jnp.int32)[:, None]
    s = jnp.arange(16, dtype=jnp.int32)[None, :]
    rolled = params[g, (s - g) % 16, :]
    pt = jnp.transpose(rolled, (2, 0, 1)).reshape(-1)
    p15 = params[:, 15, :].T.reshape(-1)
    out = _sc_call(X.reshape(-1), pt, p15)
    return out.reshape(X.shape[0], _ROW)


# speedup vs baseline: 880.5135x; 1.1345x over previous
import functools

import jax
import jax.numpy as jnp
from jax import lax
from jax.experimental import pallas as pl
from jax.experimental.pallas import tpu as pltpu
from jax.experimental.pallas import tpu_sc as plsc

_ARITY = 4
_GROUPS = 256
_OUT_DIM = 4
_BATCH = 16384
_ROW = _GROUPS * _ARITY
_N = _BATCH * _ROW
_NC = 2
_NS = 16
_NW = _NC * _NS
_SPAN = _N // _NW
_CHUNK = 16 * _ROW
_NCHUNK = _SPAN // _CHUNK
_PTWORDS = _OUT_DIM * _GROUPS * 16
_P15WORDS = _OUT_DIM * _GROUPS


def _sc_body(x_hbm, pt_hbm, p15_hbm, o_hbm, pt, p15,
             xv0, xv1, ov0, ov1, si0, si1, so0, so1):
    wid = lax.axis_index("s") * _NC + lax.axis_index("c")
    pltpu.sync_copy(pt_hbm, pt)
    pltpu.sync_copy(p15_hbm, p15)

    xvs, ovs = (xv0, xv1), (ov0, ov1)
    sis, sos = (si0, si1), (so0, so1)

    iota = jnp.arange(16, dtype=jnp.int32)
    lane4 = iota * 4
    lane16 = iota * 16
    hi_mask = jnp.int32(-16)

    base = wid * _SPAN
    pltpu.make_async_copy(x_hbm.at[pl.ds(base, _CHUNK)], xv0, si0).start()

    def chunk_pair(i, carry):
        for b in range(2):
            c = i * 2 + b
            cbase = base + c * _CHUNK
            xv, ov = xvs[b], ovs[b]

            @pl.when(c + 1 < _NCHUNK)
            def _():
                pltpu.make_async_copy(
                    x_hbm.at[pl.ds(cbase + _CHUNK, _CHUNK)],
                    xvs[1 - b], sis[1 - b]).start()

            pltpu.make_async_copy(
                x_hbm.at[pl.ds(cbase, _CHUNK)], xv, sis[b]).wait()

            @pl.when(c >= 2)
            def _():
                pltpu.make_async_copy(
                    ov, o_hbm.at[pl.ds(cbase - 2 * _CHUNK, _CHUNK)],
                    sos[b]).wait()

            @plsc.parallel_loop(0, _CHUNK // 64, unroll=4)
            def blk(t):
                i0 = t * 64 + lane4
                gb = t & 15
                gv16 = gb * 256 + lane16
                r0 = plsc.load_gather(xv, [i0])
                r1 = plsc.load_gather(xv, [i0 + 1])
                r2 = plsc.load_gather(xv, [i0 + 2])
                r3 = plsc.load_gather(xv, [i0 + 3])
                v0 = lax.bitcast_convert_type(
                    (lax.bitcast_convert_type(r0, jnp.int32) & hi_mask) | 1,
                    jnp.float32)
                v1 = lax.bitcast_convert_type(
                    (lax.bitcast_convert_type(r1, jnp.int32) & hi_mask) | 2,
                    jnp.float32)
                v2 = lax.bitcast_convert_type(
                    (lax.bitcast_convert_type(r2, jnp.int32) & hi_mask) | 4,
                    jnp.float32)
                v3 = lax.bitcast_convert_type(
                    (lax.bitcast_convert_type(r3, jnp.int32) & hi_mask) | 8,
                    jnp.float32)
                a, b = jnp.minimum(v0, v1), jnp.maximum(v0, v1)
                c2_, d2 = jnp.minimum(v2, v3), jnp.maximum(v2, v3)
                f0, s2_ = jnp.minimum(a, c2_), jnp.maximum(a, c2_)
                s1_, f3 = jnp.minimum(b, d2), jnp.maximum(b, d2)
                f1 = jnp.minimum(s1_, s2_)
                f2 = jnp.maximum(s1_, s2_)
                s1 = lax.bitcast_convert_type(f1, jnp.int32)
                s2 = lax.bitcast_convert_type(f2, jnp.int32)
                s3 = lax.bitcast_convert_type(f3, jnp.int32)
                m3 = s3 & 15
                m2 = m3 + (s2 & 15)
                m1 = m2 + (s1 & 15)
                q1 = gv16 + ((m1 + iota) & 15)
                q2 = gv16 + ((m2 + iota) & 15)
                q3 = gv16 + ((m3 + iota) & 15)
                c1 = f1 - f0
                c2c = f2 - f1
                c3 = f3 - f2
                for d in range(_OUT_DIM):
                    p15d = p15[pl.ds(d * _GROUPS + gb * 16, 16)]
                    acc0 = f0 * p15d + c1 * plsc.load_gather(pt, [q1 | (d * 4096)])
                    acc1 = c2c * plsc.load_gather(pt, [q2 | (d * 4096)]) + \
                        c3 * plsc.load_gather(pt, [q3 | (d * 4096)])
                    plsc.store_scatter(ov, [i0 + d], acc0 + acc1)

            pltpu.make_async_copy(
                ov, o_hbm.at[pl.ds(cbase, _CHUNK)], sos[b]).start()
        return carry

    lax.fori_loop(0, _NCHUNK // 2, chunk_pair, 0)
    pltpu.make_async_copy(
        ov0, o_hbm.at[pl.ds(base + (_NCHUNK - 2) * _CHUNK, _CHUNK)],
        so0).wait()
    pltpu.make_async_copy(
        ov1, o_hbm.at[pl.ds(base + (_NCHUNK - 1) * _CHUNK, _CHUNK)],
        so1).wait()


_sc_call = functools.partial(
    pl.kernel,
    mesh=plsc.VectorSubcoreMesh(core_axis_name="c", subcore_axis_name="s"),
    out_type=jax.ShapeDtypeStruct((_N,), jnp.float32),
    compiler_params=pltpu.CompilerParams(needs_layout_passes=False),
    scratch_types=[
        pltpu.VMEM((_PTWORDS,), jnp.float32),
        pltpu.VMEM((_P15WORDS,), jnp.float32),
        pltpu.VMEM((_CHUNK,), jnp.float32),
        pltpu.VMEM((_CHUNK,), jnp.float32),
        pltpu.VMEM((_CHUNK,), jnp.float32),
        pltpu.VMEM((_CHUNK,), jnp.float32),
        pltpu.SemaphoreType.DMA,
        pltpu.SemaphoreType.DMA,
        pltpu.SemaphoreType.DMA,
        pltpu.SemaphoreType.DMA,
    ],
)(_sc_body)


def kernel(X, params):
    g = jnp.arange(_GROUPS, dtype=jnp.int32)[:, None]
    s = jnp.arange(16, dtype=jnp.int32)[None, :]
    rolled = params[g, (s - g) % 16, :]
    pt = jnp.transpose(rolled, (2, 0, 1)).reshape(-1)
    p15 = params[:, 15, :].T.reshape(-1)
    out = _sc_call(X.reshape(-1), pt, p15)
    return out.reshape(X.shape[0], _ROW)
